# Initial kernel scaffold; baseline (speedup 1.0000x reference)
import jax
import jax.numpy as jnp
from jax.experimental import pallas as pl


def kernel(hidden_states, Wg, W_gate, W_up, W_down):
    raise NotImplementedError("write your pallas kernel here")


# speedup vs baseline: 1.7524x; 1.7524x over previous
import functools

import jax
import jax.numpy as jnp
from jax import lax
from jax.experimental import pallas as pl
from jax.experimental.pallas import tpu as pltpu
from jax.experimental.pallas import tpu_sc as plsc

E = 8
K = 2
D = 1024
ED = 2048
T = 2048
BLK = 256
NBLK = T * K // BLK + E
NPAD = NBLK * BLK
WL = 128
NW = 32
TPW = T // NW


def _router_body(x_ref, wg_ref, logits_ref, pos0_ref, pos1_ref,
                 w0_ref, w1_ref, eb_ref, bv_ref):
    x = x_ref[...]
    logits = lax.dot_general(x, wg_ref[...], (((1,), (1,)), ((), ())),
                             preferred_element_type=jnp.float32)
    logits_bf = logits.astype(jnp.bfloat16)
    logits_ref[...] = logits_bf
    logits = logits_bf.astype(jnp.float32)
    m = jnp.max(logits, axis=1, keepdims=True)
    p = jnp.exp(logits - m)
    probs = p / jnp.sum(p, axis=1, keepdims=True)

    idx8 = lax.broadcasted_iota(jnp.int32, (T, E), 1)
    big = jnp.int32(E)
    m1 = jnp.max(probs, axis=1, keepdims=True)
    a1 = jnp.min(jnp.where(probs == m1, idx8, big), axis=1, keepdims=True)
    probs2 = jnp.where(idx8 == a1, jnp.float32(-1.0), probs)
    m2 = jnp.max(probs2, axis=1, keepdims=True)
    a2 = jnp.min(jnp.where(probs2 == m2, idx8, big), axis=1, keepdims=True)
    tot = m1 + m2
    w1n = m1 / tot
    w2n = m2 / tot
    w0_ref[...] = jnp.broadcast_to(w1n, (T, WL))
    w1_ref[...] = jnp.broadcast_to(w2n, (T, WL))

    oh0 = (idx8 == a1).astype(jnp.float32)
    oh1 = (idx8 == a2).astype(jnp.float32)

    def icumsum(y):
        s = 1
        while s < T:
            y = y + jnp.concatenate(
                [jnp.zeros((s, E), jnp.float32), y[:T - s]], axis=0)
            s *= 2
        return y
    r0 = icumsum(oh0)
    r1 = icumsum(oh1)
    cnt0 = r0[T - 1:T, :]
    counts = cnt0 + r1[T - 1:T, :]
    cp = jnp.ceil(counts / BLK) * BLK
    tri = (lax.broadcasted_iota(jnp.int32, (E, E), 0)
           < lax.broadcasted_iota(jnp.int32, (E, E), 1)).astype(jnp.float32)
    astart = lax.dot_general(cp, tri, (((1,), (0,)), ((), ())),
                             preferred_element_type=jnp.float32)
    pos0 = jnp.sum(oh0 * (astart + r0 - 1.0), axis=1, keepdims=True)
    pos1 = jnp.sum(oh1 * (astart + cnt0 + r1 - 1.0), axis=1, keepdims=True)
    pos0_ref[...] = pos0.astype(jnp.int32)
    pos1_ref[...] = pos1.astype(jnp.int32)

    ends = astart + cp
    bs = (lax.broadcasted_iota(jnp.int32, (NBLK, 1), 0) * BLK
          ).astype(jnp.float32)
    endsb = jnp.broadcast_to(ends, (NBLK, E))
    eb = jnp.sum((endsb <= bs).astype(jnp.int32), axis=1, keepdims=True)
    total = jnp.max(ends)
    bv = (bs < total).astype(jnp.int32)
    iota8f = lax.broadcasted_iota(jnp.int32, (1, E), 1).astype(jnp.float32)
    elast = jnp.max(jnp.where(counts > 0, iota8f, 0.0)).astype(jnp.int32)
    eb_ref[...] = jnp.where(bv == 1, eb, elast)
    bv_ref[...] = bv


def _router(x, wg):
    return pl.pallas_call(
        _router_body,
        out_shape=(
            jax.ShapeDtypeStruct((T, E), jnp.bfloat16),
            jax.ShapeDtypeStruct((T, 1), jnp.int32),
            jax.ShapeDtypeStruct((T, 1), jnp.int32),
            jax.ShapeDtypeStruct((T, WL), jnp.float32),
            jax.ShapeDtypeStruct((T, WL), jnp.float32),
            jax.ShapeDtypeStruct((NBLK, 1), jnp.int32),
            jax.ShapeDtypeStruct((NBLK, 1), jnp.int32),
        ),
    )(x, wg)


def _dispatch_body(x_hbm, pos0_hbm, pos1_hbm, w0_hbm, w1_hbm,
                   xs_hbm, ws_hbm, rows_v, wrows_v, idx_v, sem):
    wid = lax.axis_index("s") * 2 + lax.axis_index("c")
    base = wid * TPW
    pltpu.sync_copy(x_hbm.at[pl.ds(base, TPW)], rows_v)
    pltpu.sync_copy(pos0_hbm.at[pl.ds(base, TPW)], idx_v)
    pltpu.async_copy(rows_v, xs_hbm.at[idx_v], sem).wait()
    pltpu.sync_copy(w0_hbm.at[pl.ds(base, TPW)], wrows_v)
    pltpu.async_copy(wrows_v, ws_hbm.at[idx_v], sem).wait()
    pltpu.sync_copy(pos1_hbm.at[pl.ds(base, TPW)], idx_v)
    pltpu.async_copy(rows_v, xs_hbm.at[idx_v], sem).wait()
    pltpu.sync_copy(w1_hbm.at[pl.ds(base, TPW)], wrows_v)
    pltpu.async_copy(wrows_v, ws_hbm.at[idx_v], sem).wait()


def _dispatch(x, pos0, pos1, w0, w1):
    mesh = plsc.VectorSubcoreMesh(core_axis_name="c", subcore_axis_name="s")
    return pl.kernel(
        _dispatch_body,
        out_type=(
            jax.ShapeDtypeStruct((NPAD, D), jnp.float32),
            jax.ShapeDtypeStruct((NPAD, WL), jnp.float32),
        ),
        mesh=mesh,
        scratch_types=[
            pltpu.VMEM((TPW, D), jnp.float32),
            pltpu.VMEM((TPW, WL), jnp.float32),
            pltpu.VMEM((TPW,), jnp.int32),
            pltpu.SemaphoreType.DMA,
        ],
    )(x, pos0, pos1, w0, w1)


def _mm_body(eb_ref, bv_ref, xs_ref, ws_ref, wg_ref, wu_ref, wd_ref, ys_ref):
    i = pl.program_id(0)

    @pl.when(bv_ref[i] == 1)
    def _():
        xb = xs_ref[...].astype(jnp.bfloat16)
        g = lax.dot_general(xb, wg_ref[0], (((1,), (1,)), ((), ())),
                            preferred_element_type=jnp.float32)
        u = lax.dot_general(xb, wu_ref[0], (((1,), (1,)), ((), ())),
                            preferred_element_type=jnp.float32)
        h = (g * jax.nn.sigmoid(g) * u).astype(jnp.bfloat16)
        y = lax.dot_general(h, wd_ref[0], (((1,), (1,)), ((), ())),
                            preferred_element_type=jnp.float32)
        ys_ref[...] = y * ws_ref[...][:, 0:1]


def _grouped_mm(eb, bv, xs, ws, wgate, wup, wdown):
    grid_spec = pltpu.PrefetchScalarGridSpec(
        num_scalar_prefetch=2,
        grid=(NBLK,),
        in_specs=[
            pl.BlockSpec((BLK, D), lambda i, eb, bv: (i, 0)),
            pl.BlockSpec((BLK, WL), lambda i, eb, bv: (i, 0)),
            pl.BlockSpec((1, ED, D), lambda i, eb, bv: (eb[i], 0, 0)),
            pl.BlockSpec((1, ED, D), lambda i, eb, bv: (eb[i], 0, 0)),
            pl.BlockSpec((1, D, ED), lambda i, eb, bv: (eb[i], 0, 0)),
        ],
        out_specs=pl.BlockSpec((BLK, D), lambda i, eb, bv: (i, 0)),
    )
    return pl.pallas_call(
        _mm_body,
        grid_spec=grid_spec,
        out_shape=jax.ShapeDtypeStruct((NPAD, D), jnp.float32),
    )(eb, bv, xs, ws, wgate, wup, wdown)


CCH = 32


def _combine_body(ys_hbm, pos0_hbm, pos1_hbm, out_hbm,
                  buf0_v, buf1_v, idx_v, sem):
    wid = lax.axis_index("s") * 2 + lax.axis_index("c")
    base = wid * TPW
    for c in range(TPW // CCH):
        cbase = base + c * CCH
        pltpu.sync_copy(pos0_hbm.at[pl.ds(cbase, CCH)], idx_v)
        pltpu.async_copy(ys_hbm.at[idx_v], buf0_v, sem).wait()
        pltpu.sync_copy(pos1_hbm.at[pl.ds(cbase, CCH)], idx_v)
        pltpu.async_copy(ys_hbm.at[idx_v], buf1_v, sem).wait()

        def add_one(i, _):
            r = i // (D // 16)
            col = (i % (D // 16)) * 16
            buf0_v[r, pl.ds(col, 16)] = (buf0_v[r, pl.ds(col, 16)]
                                         + buf1_v[r, pl.ds(col, 16)])
            return 0
        lax.fori_loop(0, CCH * (D // 16), add_one, 0)
        pltpu.sync_copy(buf0_v, out_hbm.at[pl.ds(cbase, CCH)])


def _combine(ys, pos0, pos1):
    mesh = plsc.VectorSubcoreMesh(core_axis_name="c", subcore_axis_name="s")
    return pl.kernel(
        _combine_body,
        out_type=jax.ShapeDtypeStruct((T, D), jnp.float32),
        mesh=mesh,
        scratch_types=[
            pltpu.VMEM((CCH, D), jnp.float32),
            pltpu.VMEM((CCH, D), jnp.float32),
            pltpu.VMEM((CCH,), jnp.int32),
            pltpu.SemaphoreType.DMA,
        ],
    )(ys, pos0, pos1)


def kernel(hidden_states, Wg, W_gate, W_up, W_down):
    B, S, Dm = hidden_states.shape
    x = hidden_states.reshape(S, Dm)
    logits, pos0, pos1, w0, w1, eb, bv = _router(x, Wg)
    pos0 = pos0.reshape(T)
    pos1 = pos1.reshape(T)
    xs, ws = _dispatch(x.astype(jnp.float32), pos0, pos1, w0, w1)
    ys = _grouped_mm(eb.reshape(NBLK), bv.reshape(NBLK), xs, ws,
                     W_gate, W_up, W_down)
    outf = _combine(ys, pos0, pos1)
    return outf.astype(jnp.bfloat16).reshape(B, S, Dm), logits


# speedup vs baseline: 1.9520x; 1.1139x over previous
import functools

import jax
import jax.numpy as jnp
from jax import lax
from jax.experimental import pallas as pl
from jax.experimental.pallas import tpu as pltpu
from jax.experimental.pallas import tpu_sc as plsc

E = 8
K = 2
D = 1024
ED = 2048
T = 2048
BLK = 256
NBLK = T * K // BLK + E
NPAD = NBLK * BLK
WL = 128
NW = 32
TPW = T // NW


def _router_body(x_ref, wg_ref, logits_ref, x32_ref, pos0_ref, pos1_ref,
                 w0_ref, w1_ref, eb_ref, bv_ref):
    x = x_ref[...]
    x32_ref[...] = x.astype(jnp.float32)
    logits = lax.dot_general(x, wg_ref[...], (((1,), (1,)), ((), ())),
                             preferred_element_type=jnp.float32)
    logits_bf = logits.astype(jnp.bfloat16)
    logits_ref[...] = logits_bf
    logits = logits_bf.astype(jnp.float32)
    m = jnp.max(logits, axis=1, keepdims=True)
    p = jnp.exp(logits - m)
    probs = p / jnp.sum(p, axis=1, keepdims=True)

    idx8 = lax.broadcasted_iota(jnp.int32, (T, E), 1)
    big = jnp.int32(E)
    m1 = jnp.max(probs, axis=1, keepdims=True)
    a1 = jnp.min(jnp.where(probs == m1, idx8, big), axis=1, keepdims=True)
    probs2 = jnp.where(idx8 == a1, jnp.float32(-1.0), probs)
    m2 = jnp.max(probs2, axis=1, keepdims=True)
    a2 = jnp.min(jnp.where(probs2 == m2, idx8, big), axis=1, keepdims=True)
    tot = m1 + m2
    w1n = m1 / tot
    w2n = m2 / tot
    w0_ref[...] = jnp.broadcast_to(w1n, (T, WL))
    w1_ref[...] = jnp.broadcast_to(w2n, (T, WL))

    oh0 = (idx8 == a1).astype(jnp.float32)
    oh1 = (idx8 == a2).astype(jnp.float32)

    def icumsum(y):
        s = 1
        while s < T:
            y = y + jnp.concatenate(
                [jnp.zeros((s, E), jnp.float32), y[:T - s]], axis=0)
            s *= 2
        return y
    r0 = icumsum(oh0)
    r1 = icumsum(oh1)
    cnt0 = r0[T - 1:T, :]
    counts = cnt0 + r1[T - 1:T, :]
    cp = jnp.ceil(counts / BLK) * BLK
    tri = (lax.broadcasted_iota(jnp.int32, (E, E), 0)
           < lax.broadcasted_iota(jnp.int32, (E, E), 1)).astype(jnp.float32)
    astart = lax.dot_general(cp, tri, (((1,), (0,)), ((), ())),
                             preferred_element_type=jnp.float32)
    pos0 = jnp.sum(oh0 * (astart + r0 - 1.0), axis=1, keepdims=True)
    pos1 = jnp.sum(oh1 * (astart + cnt0 + r1 - 1.0), axis=1, keepdims=True)
    pos0_ref[...] = pos0.astype(jnp.int32)
    pos1_ref[...] = pos1.astype(jnp.int32)

    ends = astart + cp
    bs = (lax.broadcasted_iota(jnp.int32, (NBLK, 1), 0) * BLK
          ).astype(jnp.float32)
    endsb = jnp.broadcast_to(ends, (NBLK, E))
    eb = jnp.sum((endsb <= bs).astype(jnp.int32), axis=1, keepdims=True)
    total = jnp.max(ends)
    bv = (bs < total).astype(jnp.int32)
    iota8f = lax.broadcasted_iota(jnp.int32, (1, E), 1).astype(jnp.float32)
    elast = jnp.max(jnp.where(counts > 0, iota8f, 0.0)).astype(jnp.int32)
    eb_ref[...] = jnp.where(bv == 1, eb, elast)
    bv_ref[...] = bv


def _router(x, wg):
    return pl.pallas_call(
        _router_body,
        out_shape=(
            jax.ShapeDtypeStruct((T, E), jnp.bfloat16),
            jax.ShapeDtypeStruct((T, D), jnp.float32),
            jax.ShapeDtypeStruct((T, 1), jnp.int32),
            jax.ShapeDtypeStruct((T, 1), jnp.int32),
            jax.ShapeDtypeStruct((T, WL), jnp.float32),
            jax.ShapeDtypeStruct((T, WL), jnp.float32),
            jax.ShapeDtypeStruct((NBLK, 1), jnp.int32),
            jax.ShapeDtypeStruct((NBLK, 1), jnp.int32),
        ),
    )(x, wg)


def _dispatch_body(x_hbm, pos0_hbm, pos1_hbm, w0_hbm, w1_hbm,
                   xs_hbm, ws_hbm, rows_v, wrows_v, idx_v, sem):
    wid = lax.axis_index("s") * 2 + lax.axis_index("c")
    base = wid * TPW
    pltpu.sync_copy(x_hbm.at[pl.ds(base, TPW)], rows_v)
    pltpu.sync_copy(pos0_hbm.at[pl.ds(base, TPW)], idx_v)
    pltpu.async_copy(rows_v, xs_hbm.at[idx_v], sem).wait()
    pltpu.sync_copy(w0_hbm.at[pl.ds(base, TPW)], wrows_v)
    pltpu.async_copy(wrows_v, ws_hbm.at[idx_v], sem).wait()
    pltpu.sync_copy(pos1_hbm.at[pl.ds(base, TPW)], idx_v)
    pltpu.async_copy(rows_v, xs_hbm.at[idx_v], sem).wait()
    pltpu.sync_copy(w1_hbm.at[pl.ds(base, TPW)], wrows_v)
    pltpu.async_copy(wrows_v, ws_hbm.at[idx_v], sem).wait()


def _dispatch(x, pos0, pos1, w0, w1):
    mesh = plsc.VectorSubcoreMesh(core_axis_name="c", subcore_axis_name="s")
    return pl.kernel(
        _dispatch_body,
        out_type=(
            jax.ShapeDtypeStruct((NPAD, D), jnp.float32),
            jax.ShapeDtypeStruct((NPAD, WL), jnp.float32),
        ),
        mesh=mesh,
        scratch_types=[
            pltpu.VMEM((TPW, D), jnp.float32),
            pltpu.VMEM((TPW, WL), jnp.float32),
            pltpu.VMEM((TPW,), jnp.int32),
            pltpu.SemaphoreType.DMA,
        ],
    )(x, pos0, pos1, w0, w1)


def _mm_body(eb_ref, bv_ref, xs_ref, ws_ref, wg_ref, wu_ref, wd_ref, ys_ref):
    i = pl.program_id(0)

    @pl.when(bv_ref[i] == 1)
    def _():
        xb = xs_ref[...].astype(jnp.bfloat16)
        g = lax.dot_general(xb, wg_ref[0], (((1,), (1,)), ((), ())),
                            preferred_element_type=jnp.float32)
        u = lax.dot_general(xb, wu_ref[0], (((1,), (1,)), ((), ())),
                            preferred_element_type=jnp.float32)
        h = (g * jax.nn.sigmoid(g) * u).astype(jnp.bfloat16)
        y = lax.dot_general(h, wd_ref[0], (((1,), (1,)), ((), ())),
                            preferred_element_type=jnp.float32)
        ys_ref[...] = y * ws_ref[...][:, 0:1]


def _grouped_mm(eb, bv, xs, ws, wgate, wup, wdown):
    grid_spec = pltpu.PrefetchScalarGridSpec(
        num_scalar_prefetch=2,
        grid=(NBLK,),
        in_specs=[
            pl.BlockSpec((BLK, D), lambda i, eb, bv: (i, 0)),
            pl.BlockSpec((BLK, WL), lambda i, eb, bv: (i, 0)),
            pl.BlockSpec((1, ED, D), lambda i, eb, bv: (eb[i], 0, 0)),
            pl.BlockSpec((1, ED, D), lambda i, eb, bv: (eb[i], 0, 0)),
            pl.BlockSpec((1, D, ED), lambda i, eb, bv: (eb[i], 0, 0)),
        ],
        out_specs=pl.BlockSpec((BLK, D), lambda i, eb, bv: (i, 0)),
    )
    return pl.pallas_call(
        _mm_body,
        grid_spec=grid_spec,
        out_shape=jax.ShapeDtypeStruct((NPAD, D), jnp.float32),
    )(eb, bv, xs, ws, wgate, wup, wdown)


CCH = 32


def _combine_body(ys_hbm, pos0_hbm, pos1_hbm, out_hbm,
                  buf0_v, buf1_v, idx_v, sem):
    wid = lax.axis_index("s") * 2 + lax.axis_index("c")
    base = wid * TPW
    for c in range(TPW // CCH):
        cbase = base + c * CCH
        pltpu.sync_copy(pos0_hbm.at[pl.ds(cbase, CCH)], idx_v)
        pltpu.async_copy(ys_hbm.at[idx_v], buf0_v, sem).wait()
        pltpu.sync_copy(pos1_hbm.at[pl.ds(cbase, CCH)], idx_v)
        pltpu.async_copy(ys_hbm.at[idx_v], buf1_v, sem).wait()

        @plsc.parallel_loop(0, CCH * (D // 16), 1, unroll=8)
        def _add(i):
            r = i // (D // 16)
            col = (i % (D // 16)) * 16
            buf0_v[r, pl.ds(col, 16)] = (buf0_v[r, pl.ds(col, 16)]
                                         + buf1_v[r, pl.ds(col, 16)])
        pltpu.sync_copy(buf0_v, out_hbm.at[pl.ds(cbase, CCH)])


def _combine(ys, pos0, pos1):
    mesh = plsc.VectorSubcoreMesh(core_axis_name="c", subcore_axis_name="s")
    return pl.kernel(
        _combine_body,
        out_type=jax.ShapeDtypeStruct((T, D), jnp.float32),
        mesh=mesh,
        scratch_types=[
            pltpu.VMEM((CCH, D), jnp.float32),
            pltpu.VMEM((CCH, D), jnp.float32),
            pltpu.VMEM((CCH,), jnp.int32),
            pltpu.SemaphoreType.DMA,
        ],
    )(ys, pos0, pos1)


def kernel(hidden_states, Wg, W_gate, W_up, W_down):
    B, S, Dm = hidden_states.shape
    x = hidden_states.reshape(S, Dm)
    logits, x32, pos0, pos1, w0, w1, eb, bv = _router(x, Wg)
    pos0 = pos0.reshape(T)
    pos1 = pos1.reshape(T)
    xs, ws = _dispatch(x32, pos0, pos1, w0, w1)
    ys = _grouped_mm(eb.reshape(NBLK), bv.reshape(NBLK), xs, ws,
                     W_gate, W_up, W_down)
    outf = _combine(ys, pos0, pos1)
    return outf.astype(jnp.bfloat16).reshape(B, S, Dm), logits


# speedup vs baseline: 1.9697x; 1.0091x over previous
import functools

import jax
import jax.numpy as jnp
from jax import lax
from jax.experimental import pallas as pl
from jax.experimental.pallas import tpu as pltpu
from jax.experimental.pallas import tpu_sc as plsc

E = 8
K = 2
D = 1024
ED = 2048
T = 2048
BLK = 256
NBLK = T * K // BLK + E
NPAD = NBLK * BLK
WL = 16
NW = 32
TPW = T // NW


def _router_body(x_ref, wg_ref, logits_ref, x32_ref, pos0_ref, pos1_ref,
                 w0_ref, w1_ref, eb_ref, bv_ref):
    x = x_ref[...]
    x32_ref[...] = x.astype(jnp.float32)
    logits = lax.dot_general(x, wg_ref[...], (((1,), (1,)), ((), ())),
                             preferred_element_type=jnp.float32)
    logits_bf = logits.astype(jnp.bfloat16)
    logits_ref[...] = logits_bf
    logits = logits_bf.astype(jnp.float32)
    m = jnp.max(logits, axis=1, keepdims=True)
    p = jnp.exp(logits - m)
    probs = p / jnp.sum(p, axis=1, keepdims=True)

    idx8 = lax.broadcasted_iota(jnp.int32, (T, E), 1)
    big = jnp.int32(E)
    m1 = jnp.max(probs, axis=1, keepdims=True)
    a1 = jnp.min(jnp.where(probs == m1, idx8, big), axis=1, keepdims=True)
    probs2 = jnp.where(idx8 == a1, jnp.float32(-1.0), probs)
    m2 = jnp.max(probs2, axis=1, keepdims=True)
    a2 = jnp.min(jnp.where(probs2 == m2, idx8, big), axis=1, keepdims=True)
    tot = m1 + m2
    w1n = m1 / tot
    w2n = m2 / tot
    w0_ref[...] = jnp.broadcast_to(w1n, (T, WL))
    w1_ref[...] = jnp.broadcast_to(w2n, (T, WL))

    oh0 = (idx8 == a1).astype(jnp.float32)
    oh1 = (idx8 == a2).astype(jnp.float32)

    def icumsum(y):
        s = 1
        while s < T:
            y = y + jnp.concatenate(
                [jnp.zeros((s, E), jnp.float32), y[:T - s]], axis=0)
            s *= 2
        return y
    r0 = icumsum(oh0)
    r1 = icumsum(oh1)
    cnt0 = r0[T - 1:T, :]
    counts = cnt0 + r1[T - 1:T, :]
    cp = jnp.ceil(counts / BLK) * BLK
    tri = (lax.broadcasted_iota(jnp.int32, (E, E), 0)
           < lax.broadcasted_iota(jnp.int32, (E, E), 1)).astype(jnp.float32)
    astart = lax.dot_general(cp, tri, (((1,), (0,)), ((), ())),
                             preferred_element_type=jnp.float32)
    pos0 = jnp.sum(oh0 * (astart + r0 - 1.0), axis=1, keepdims=True)
    pos1 = jnp.sum(oh1 * (astart + cnt0 + r1 - 1.0), axis=1, keepdims=True)
    pos0_ref[...] = pos0.astype(jnp.int32)
    pos1_ref[...] = pos1.astype(jnp.int32)

    ends = astart + cp
    bs = (lax.broadcasted_iota(jnp.int32, (NBLK, 1), 0) * BLK
          ).astype(jnp.float32)
    endsb = jnp.broadcast_to(ends, (NBLK, E))
    eb = jnp.sum((endsb <= bs).astype(jnp.int32), axis=1, keepdims=True)
    total = jnp.max(ends)
    bv = (bs < total).astype(jnp.int32)
    iota8f = lax.broadcasted_iota(jnp.int32, (1, E), 1).astype(jnp.float32)
    elast = jnp.max(jnp.where(counts > 0, iota8f, 0.0)).astype(jnp.int32)
    eb_ref[...] = jnp.where(bv == 1, eb, elast)
    bv_ref[...] = bv


def _router(x, wg):
    return pl.pallas_call(
        _router_body,
        out_shape=(
            jax.ShapeDtypeStruct((T, E), jnp.bfloat16),
            jax.ShapeDtypeStruct((T, D), jnp.float32),
            jax.ShapeDtypeStruct((T, 1), jnp.int32),
            jax.ShapeDtypeStruct((T, 1), jnp.int32),
            jax.ShapeDtypeStruct((T, WL), jnp.float32),
            jax.ShapeDtypeStruct((T, WL), jnp.float32),
            jax.ShapeDtypeStruct((NBLK, 1), jnp.int32),
            jax.ShapeDtypeStruct((NBLK, 1), jnp.int32),
        ),
    )(x, wg)


def _dispatch_body(x_hbm, pos0_hbm, pos1_hbm, xs_hbm, rows_v, idx_v, sem):
    wid = lax.axis_index("s") * 2 + lax.axis_index("c")
    base = wid * TPW
    pltpu.sync_copy(x_hbm.at[pl.ds(base, TPW)], rows_v)
    pltpu.sync_copy(pos0_hbm.at[pl.ds(base, TPW)], idx_v)
    pltpu.async_copy(rows_v, xs_hbm.at[idx_v], sem).wait()
    pltpu.sync_copy(pos1_hbm.at[pl.ds(base, TPW)], idx_v)
    pltpu.async_copy(rows_v, xs_hbm.at[idx_v], sem).wait()


def _dispatch(x, pos0, pos1):
    mesh = plsc.VectorSubcoreMesh(core_axis_name="c", subcore_axis_name="s")
    return pl.kernel(
        _dispatch_body,
        out_type=jax.ShapeDtypeStruct((NPAD, D), jnp.float32),
        mesh=mesh,
        scratch_types=[
            pltpu.VMEM((TPW, D), jnp.float32),
            pltpu.VMEM((TPW,), jnp.int32),
            pltpu.SemaphoreType.DMA,
        ],
    )(x, pos0, pos1)


def _mm_body(eb_ref, bv_ref, xs_ref, wg_ref, wu_ref, wd_ref, ys_ref):
    i = pl.program_id(0)

    @pl.when(bv_ref[i] == 1)
    def _():
        xb = xs_ref[...].astype(jnp.bfloat16)
        g = lax.dot_general(xb, wg_ref[0], (((1,), (1,)), ((), ())),
                            preferred_element_type=jnp.float32)
        u = lax.dot_general(xb, wu_ref[0], (((1,), (1,)), ((), ())),
                            preferred_element_type=jnp.float32)
        h = (g * jax.nn.sigmoid(g) * u).astype(jnp.bfloat16)
        ys_ref[...] = lax.dot_general(h, wd_ref[0], (((1,), (1,)), ((), ())),
                                      preferred_element_type=jnp.float32)


def _grouped_mm(eb, bv, xs, wgate, wup, wdown):
    grid_spec = pltpu.PrefetchScalarGridSpec(
        num_scalar_prefetch=2,
        grid=(NBLK,),
        in_specs=[
            pl.BlockSpec((BLK, D), lambda i, eb, bv: (i, 0)),
            pl.BlockSpec((1, ED, D), lambda i, eb, bv: (eb[i], 0, 0)),
            pl.BlockSpec((1, ED, D), lambda i, eb, bv: (eb[i], 0, 0)),
            pl.BlockSpec((1, D, ED), lambda i, eb, bv: (eb[i], 0, 0)),
        ],
        out_specs=pl.BlockSpec((BLK, D), lambda i, eb, bv: (i, 0)),
    )
    return pl.pallas_call(
        _mm_body,
        grid_spec=grid_spec,
        out_shape=jax.ShapeDtypeStruct((NPAD, D), jnp.float32),
    )(eb, bv, xs, wgate, wup, wdown)


CCH = 32


def _combine_body(ys_hbm, pos0_hbm, pos1_hbm, w0_hbm, w1_hbm, out_hbm,
                  buf0_v, buf1_v, wv0, wv1, idx_v, sem):
    wid = lax.axis_index("s") * 2 + lax.axis_index("c")
    base = wid * TPW
    for c in range(TPW // CCH):
        cbase = base + c * CCH
        pltpu.sync_copy(pos0_hbm.at[pl.ds(cbase, CCH)], idx_v)
        pltpu.async_copy(ys_hbm.at[idx_v], buf0_v, sem).wait()
        pltpu.sync_copy(pos1_hbm.at[pl.ds(cbase, CCH)], idx_v)
        pltpu.async_copy(ys_hbm.at[idx_v], buf1_v, sem).wait()
        pltpu.sync_copy(w0_hbm.at[pl.ds(cbase, CCH)], wv0)
        pltpu.sync_copy(w1_hbm.at[pl.ds(cbase, CCH)], wv1)

        @plsc.parallel_loop(0, CCH * (D // 16), 1, unroll=8)
        def _add(i):
            r = i // (D // 16)
            col = (i % (D // 16)) * 16
            buf0_v[r, pl.ds(col, 16)] = (
                buf0_v[r, pl.ds(col, 16)] * wv0[r, pl.ds(0, 16)]
                + buf1_v[r, pl.ds(col, 16)] * wv1[r, pl.ds(0, 16)])
        pltpu.sync_copy(buf0_v, out_hbm.at[pl.ds(cbase, CCH)])


def _combine(ys, pos0, pos1, w0, w1):
    mesh = plsc.VectorSubcoreMesh(core_axis_name="c", subcore_axis_name="s")
    return pl.kernel(
        _combine_body,
        out_type=jax.ShapeDtypeStruct((T, D), jnp.float32),
        mesh=mesh,
        scratch_types=[
            pltpu.VMEM((CCH, D), jnp.float32),
            pltpu.VMEM((CCH, D), jnp.float32),
            pltpu.VMEM((CCH, WL), jnp.float32),
            pltpu.VMEM((CCH, WL), jnp.float32),
            pltpu.VMEM((CCH,), jnp.int32),
            pltpu.SemaphoreType.DMA,
        ],
    )(ys, pos0, pos1, w0, w1)


def kernel(hidden_states, Wg, W_gate, W_up, W_down):
    B, S, Dm = hidden_states.shape
    x = hidden_states.reshape(S, Dm)
    logits, x32, pos0, pos1, w0, w1, eb, bv = _router(x, Wg)
    pos0 = pos0.reshape(T)
    pos1 = pos1.reshape(T)
    xs = _dispatch(x32, pos0, pos1)
    ys = _grouped_mm(eb.reshape(NBLK), bv.reshape(NBLK), xs,
                     W_gate, W_up, W_down)
    outf = _combine(ys, pos0, pos1, w0, w1)
    return outf.astype(jnp.bfloat16).reshape(B, S, Dm), logits
